# baseline (speedup 1.0000x reference)
import functools
import math

import jax
import jax.numpy as jnp
from jax import lax
from jax.experimental import pallas as pl
from jax.experimental.pallas import tpu as pltpu
from jax.experimental.pallas import tpu_sc as plsc

N = 10000
E = 320000
DI = 128
KP = 32
BZ = 100
NPG = 100
ER = 3200
EC = 100
ECA = 50
N2 = 10240
NC = 2
NS = 16
BLK = 2000


def _sc_mesh():
    return plsc.VectorSubcoreMesh(core_axis_name="c", subcore_axis_name="s")


def _sc_degrees(ei4, zeros_hist, ones_ec):
    rows_per_tile = ER // NS

    @functools.partial(
        pl.kernel,
        out_type=jax.ShapeDtypeStruct((2 * N2,), jnp.float32),
        mesh=_sc_mesh(),
        scratch_types=[
            pltpu.VMEM((rows_per_tile, EC), jnp.int32),
            pltpu.VMEM((EC,), jnp.float32),
            pltpu.VMEM_SHARED((N2,), jnp.float32),
            pltpu.SemaphoreType.DMA,
        ],
    )
    def deg_k(ei_hbm, z_hbm, one_hbm, out_hbm, idx_v, ones_v, hist_sh, sem):
        c = lax.axis_index("c")
        s = lax.axis_index("s")
        seg = N2 // NS
        pltpu.sync_copy(z_hbm.at[pl.ds(s * seg, seg)], hist_sh.at[pl.ds(s * seg, seg)])
        pltpu.sync_copy(one_hbm, ones_v)
        pltpu.sync_copy(ei_hbm.at[c, s], idx_v)
        plsc.subcore_barrier()

        K = 8

        @pl.loop(0, rows_per_tile, step=K)
        def _(j):
            for b in range(K):
                pltpu.async_copy(ones_v, hist_sh.at[idx_v.at[j + b]], sem, add=True)
            for b in range(K):
                pltpu.make_async_copy(ones_v, hist_sh.at[idx_v.at[0]], sem).wait()

        plsc.subcore_barrier()
        pltpu.sync_copy(hist_sh.at[pl.ds(s * seg, seg)],
                        out_hbm.at[pl.ds(c * N2 + s * seg, seg)])

    return deg_k(ei4, zeros_hist, ones_ec).reshape(2, N2)


def _sc_agg(hn, ei5, zeros_rows):
    seg = 624
    PH = 10
    PH_ROWS = 20
    NB = 6

    @functools.partial(
        pl.kernel,
        out_type=jax.ShapeDtypeStruct((NC, N, DI), jnp.float32),
        mesh=_sc_mesh(),
        scratch_types=[
            pltpu.VMEM((PH_ROWS, ECA), jnp.int32),
            pltpu.VMEM((PH_ROWS, ECA), jnp.int32),
        ] + [pltpu.VMEM((ECA, DI), jnp.float32)] * NB
          + [pltpu.VMEM_SHARED((N, DI), jnp.float32)]
          + [pltpu.SemaphoreType.DMA] * (2 * NB),
    )
    def agg_k(hn_hbm, ei_hbm, z_hbm, out_hbm, idx_s, idx_d, *rest):
        rows = rest[:NB]
        agg_sh = rest[NB]
        gsem = rest[NB + 1:NB + 1 + NB]
        ssem = rest[NB + 1 + NB:]
        c = lax.axis_index("c")
        s = lax.axis_index("s")
        w = c * NS + s
        pltpu.sync_copy(z_hbm, agg_sh.at[pl.ds(s * seg, seg)])

        @pl.when(s == NS - 1)
        def _():
            pltpu.sync_copy(z_hbm.at[pl.ds(0, N - NS * seg)],
                            agg_sh.at[pl.ds(NS * seg, N - NS * seg)])

        plsc.subcore_barrier()

        def fire_gather(b, jj):
            pltpu.async_copy(hn_hbm.at[idx_s.at[jj]], rows[b], gsem[b])

        def wait_gather(b):
            pltpu.make_async_copy(hn_hbm.at[idx_s.at[0]], rows[b], gsem[b]).wait()

        def fire_scatter(b, jj):
            pltpu.async_copy(rows[b], agg_sh.at[idx_d.at[jj]], ssem[b], add=True)

        def wait_scatter(b):
            pltpu.make_async_copy(rows[b], agg_sh.at[idx_d.at[0]], ssem[b]).wait()

        for p in range(PH):
            g = PH * w + p
            pltpu.sync_copy(ei_hbm.at[0, g], idx_s)
            pltpu.sync_copy(ei_hbm.at[1, g], idx_d)
            for b in range(min(NB, PH_ROWS)):
                fire_gather(b, b)

            @pl.loop(0, PH_ROWS, step=NB)
            def _(j):
                for b in range(NB):
                    @pl.when(j + b < PH_ROWS)
                    def _(b=b):
                        wait_gather(b)
                        fire_scatter(b, j + b)
                for b in range(NB):
                    @pl.when(j + b + NB < PH_ROWS)
                    def _(b=b):
                        wait_scatter(b)
                        fire_gather(b, j + b + NB)

            for b in range(min(NB, PH_ROWS)):
                wait_scatter(b)

        plsc.subcore_barrier()
        pltpu.sync_copy(agg_sh.at[pl.ds(s * seg, seg)], out_hbm.at[c, pl.ds(s * seg, seg)])

        @pl.when(s == NS - 1)
        def _():
            pltpu.sync_copy(agg_sh.at[pl.ds(NS * seg, N - NS * seg)],
                            out_hbm.at[c, pl.ds(NS * seg, N - NS * seg)])

    return agg_k(hn, ei5, zeros_rows)


def _prescale(h, deg_o):
    def body(h_ref, d_ref, o_ref):
        d = d_ref[...]
        cs = jnp.where(d > 0, lax.rsqrt(jnp.maximum(d, 1.0)), 1.0)
        o_ref[...] = h_ref[...] * cs

    return pl.pallas_call(
        body, out_shape=jax.ShapeDtypeStruct((N, DI), jnp.float32)
    )(h, deg_o)


def _dense_fused(parts, deg_i, h, W_f, b_f, W_p, b_p, g_f, bt_f, g_p, bt_p):
    P1 = N // BLK
    P2 = N // BLK
    M = BZ * KP
    RB = 800
    P3 = M // RB
    GPB = BLK // NPG
    inv_scale = 1.0 / math.sqrt(float(M))

    def body(p0_r, p1_r, d_r, h_r, wf_r, bf_r, wp_r, bp_r, gf_r, btf_r, gp_r,
             btp_r, o_r, of_s, op_s, pool_s, sf_s, sp_s):
        i = pl.program_id(0)

        @pl.when(i < P1)
        def _():
            d = d_r[...]
            cd = jnp.where(d > 0, lax.rsqrt(jnp.maximum(d, 1.0)), 1.0)
            agg = (p0_r[0] + p1_r[0]) * cd
            of = jnp.dot(agg, wf_r[...], preferred_element_type=jnp.float32) + bf_r[...]
            op = jnp.dot(agg, wp_r[...], preferred_element_type=jnp.float32) + bp_r[...]
            of_s[pl.ds(i * BLK, BLK), :] = of
            op_s[pl.ds(i * BLK, BLK), :] = op

            @pl.when(i == 0)
            def _():
                sf_s[...] = jnp.zeros_like(sf_s)
                sp_s[...] = jnp.zeros_like(sp_s)

            sf_s[...] += jnp.concatenate(
                [jnp.sum(of, axis=0, keepdims=True),
                 jnp.sum(of * of, axis=0, keepdims=True)], axis=0)
            sp_s[...] += jnp.concatenate(
                [jnp.sum(op, axis=0, keepdims=True),
                 jnp.sum(op * op, axis=0, keepdims=True)], axis=0)

        @pl.when(jnp.logical_and(i >= P1, i < P1 + P2))
        def _():
            ib = i - P1
            inv_n = 1.0 / N
            sf = sf_s[...]
            mean_f = sf[0:1] * inv_n
            var_f = sf[1:2] * inv_n - mean_f * mean_f
            of = of_s[pl.ds(ib * BLK, BLK), :]
            feat = (of - mean_f) * lax.rsqrt(var_f + 1e-5) * gf_r[...] + btf_r[...]
            feat = jnp.maximum(feat, 0.0) + h_r[...]
            sp_ = sp_s[...]
            mean_p = sp_[0:1] * inv_n
            var_p = sp_[1:2] * inv_n - mean_p * mean_p
            op = op_s[pl.ds(ib * BLK, BLK), :]
            a = (op - mean_p) * lax.rsqrt(var_p + 1e-5) * gp_r[...] + btp_r[...]
            a = jnp.maximum(a, 0.0)
            mx = jnp.max(a, axis=1, keepdims=True)
            ex = jnp.exp(a - mx)
            sm = ex / jnp.sum(ex, axis=1, keepdims=True)
            for g in range(GPB):
                ag = sm[g * NPG:(g + 1) * NPG]
                fg = feat[g * NPG:(g + 1) * NPG]
                pg = lax.dot_general(ag, fg, (((0,), (0,)), ((), ())),
                                     preferred_element_type=jnp.float32)
                pool_s[pl.ds(ib * GPB * KP + g * KP, KP), :] = pg.astype(jnp.bfloat16)

        @pl.when(i >= P1 + P2)
        def _():
            ob = i - (P1 + P2)
            pr = pool_s[pl.ds(ob * RB, RB), :]
            o_r[...] = lax.dot_general(
                pr, pool_s[...], (((1,), (1,)), ((), ())),
                preferred_element_type=jnp.float32) * inv_scale

    c0 = lambda i: (0, 0)
    return pl.pallas_call(
        body,
        grid=(P1 + P2 + P3,),
        in_specs=[
            pl.BlockSpec((1, BLK, DI), lambda i: (0, jnp.minimum(i, P1 - 1), 0)),
            pl.BlockSpec((1, BLK, DI), lambda i: (1, jnp.minimum(i, P1 - 1), 0)),
            pl.BlockSpec((BLK, 1), lambda i: (jnp.minimum(i, P1 - 1), 0)),
            pl.BlockSpec((BLK, DI),
                         lambda i: (jnp.clip(i - P1, 0, P2 - 1), 0)),
            pl.BlockSpec((DI, DI), c0),
            pl.BlockSpec((1, DI), c0),
            pl.BlockSpec((DI, KP), c0),
            pl.BlockSpec((1, KP), c0),
            pl.BlockSpec((1, DI), c0),
            pl.BlockSpec((1, DI), c0),
            pl.BlockSpec((1, KP), c0),
            pl.BlockSpec((1, KP), c0),
        ],
        out_specs=pl.BlockSpec((RB, M),
                               lambda i: (jnp.clip(i - P1 - P2, 0, P3 - 1), 0)),
        out_shape=jax.ShapeDtypeStruct((M, M), jnp.float32),
        scratch_shapes=[
            pltpu.VMEM((N, DI), jnp.float32),
            pltpu.VMEM((N, KP), jnp.float32),
            pltpu.VMEM((M, DI), jnp.bfloat16),
            pltpu.VMEM((2, DI), jnp.float32),
            pltpu.VMEM((2, KP), jnp.float32),
        ],
    )(parts, parts, deg_i, h, W_f, b_f, W_p, b_p, g_f, bt_f, g_p, bt_p)


def kernel(h, edge_index, e, W_feat, b_feat, gamma_feat, beta_feat,
           W_pool, b_pool, gamma_pool, beta_pool):
    del e
    ei4 = edge_index.reshape(2, NS, ER // NS, EC)
    ei5 = edge_index.reshape(2, 10 * NC * NS, 20, ECA)
    zeros_hist = jnp.zeros((N2,), jnp.float32)
    ones_ec = jnp.ones((EC,), jnp.float32)
    deg = _sc_degrees(ei4, zeros_hist, ones_ec)
    deg_o = deg[0, :N].reshape(N, 1)
    deg_i = deg[1, :N].reshape(N, 1)
    hn = _prescale(h, deg_o)
    zeros_rows = jnp.zeros((624, DI), jnp.float32)
    parts = _sc_agg(hn, ei5, zeros_rows)
    return _dense_fused(parts, deg_i, h, W_feat, b_feat.reshape(1, DI),
                        W_pool, b_pool.reshape(1, KP),
                        gamma_feat.reshape(1, DI), beta_feat.reshape(1, DI),
                        gamma_pool.reshape(1, KP), beta_pool.reshape(1, KP))


# speedup vs baseline: 1.0482x; 1.0482x over previous
import functools
import math

import jax
import jax.numpy as jnp
from jax import lax
from jax.experimental import pallas as pl
from jax.experimental.pallas import tpu as pltpu
from jax.experimental.pallas import tpu_sc as plsc

N = 10000
E = 320000
DI = 128
KP = 32
BZ = 100
NPG = 100
ER = 3200
EC = 100
ECA = 80
N2 = 10240
NC = 2
NS = 16
BLK = 2000


def _sc_mesh():
    return plsc.VectorSubcoreMesh(core_axis_name="c", subcore_axis_name="s")


def _sc_degrees(ei4, zeros_hist, ones_ec):
    rows_per_tile = ER // NS

    @functools.partial(
        pl.kernel,
        out_type=jax.ShapeDtypeStruct((2 * N2,), jnp.float32),
        mesh=_sc_mesh(),
        scratch_types=[
            pltpu.VMEM((rows_per_tile, EC), jnp.int32),
            pltpu.VMEM((EC,), jnp.float32),
            pltpu.VMEM_SHARED((N2,), jnp.float32),
            pltpu.SemaphoreType.DMA,
        ],
    )
    def deg_k(ei_hbm, z_hbm, one_hbm, out_hbm, idx_v, ones_v, hist_sh, sem):
        c = lax.axis_index("c")
        s = lax.axis_index("s")
        seg = N2 // NS
        pltpu.sync_copy(z_hbm.at[pl.ds(s * seg, seg)], hist_sh.at[pl.ds(s * seg, seg)])
        pltpu.sync_copy(one_hbm, ones_v)
        pltpu.sync_copy(ei_hbm.at[c, s], idx_v)
        plsc.subcore_barrier()

        K = 8

        @pl.loop(0, rows_per_tile, step=K)
        def _(j):
            for b in range(K):
                pltpu.async_copy(ones_v, hist_sh.at[idx_v.at[j + b]], sem, add=True)
            for b in range(K):
                pltpu.make_async_copy(ones_v, hist_sh.at[idx_v.at[0]], sem).wait()

        plsc.subcore_barrier()
        pltpu.sync_copy(hist_sh.at[pl.ds(s * seg, seg)],
                        out_hbm.at[pl.ds(c * N2 + s * seg, seg)])

    return deg_k(ei4, zeros_hist, ones_ec).reshape(2, N2)


def _sc_agg(hn, ei5, zeros_rows):
    seg = 624
    PH = 5
    PH_ROWS = 25
    NB = 4

    @functools.partial(
        pl.kernel,
        out_type=jax.ShapeDtypeStruct((NC, N, DI), jnp.float32),
        mesh=_sc_mesh(),
        scratch_types=[
            pltpu.VMEM((PH_ROWS, ECA), jnp.int32),
            pltpu.VMEM((PH_ROWS, ECA), jnp.int32),
        ] + [pltpu.VMEM((ECA, DI), jnp.float32)] * NB
          + [pltpu.VMEM_SHARED((N, DI), jnp.float32)]
          + [pltpu.SemaphoreType.DMA] * (2 * NB),
    )
    def agg_k(hn_hbm, ei_hbm, z_hbm, out_hbm, idx_s, idx_d, *rest):
        rows = rest[:NB]
        agg_sh = rest[NB]
        gsem = rest[NB + 1:NB + 1 + NB]
        ssem = rest[NB + 1 + NB:]
        c = lax.axis_index("c")
        s = lax.axis_index("s")
        w = c * NS + s
        pltpu.sync_copy(z_hbm, agg_sh.at[pl.ds(s * seg, seg)])

        @pl.when(s == NS - 1)
        def _():
            pltpu.sync_copy(z_hbm.at[pl.ds(0, N - NS * seg)],
                            agg_sh.at[pl.ds(NS * seg, N - NS * seg)])

        plsc.subcore_barrier()

        def fire_gather(b, jj):
            pltpu.async_copy(hn_hbm.at[idx_s.at[jj]], rows[b], gsem[b])

        def wait_gather(b):
            pltpu.make_async_copy(hn_hbm.at[idx_s.at[0]], rows[b], gsem[b]).wait()

        def fire_scatter(b, jj):
            pltpu.async_copy(rows[b], agg_sh.at[idx_d.at[jj]], ssem[b], add=True)

        def wait_scatter(b):
            pltpu.make_async_copy(rows[b], agg_sh.at[idx_d.at[0]], ssem[b]).wait()

        for p in range(PH):
            g = PH * w + p
            pltpu.sync_copy(ei_hbm.at[0, g], idx_s)
            pltpu.sync_copy(ei_hbm.at[1, g], idx_d)
            for b in range(min(NB, PH_ROWS)):
                fire_gather(b, b)

            @pl.loop(0, PH_ROWS, step=NB)
            def _(j):
                for b in range(NB):
                    @pl.when(j + b < PH_ROWS)
                    def _(b=b):
                        wait_gather(b)
                        fire_scatter(b, j + b)
                for b in range(NB):
                    @pl.when(j + b + NB < PH_ROWS)
                    def _(b=b):
                        wait_scatter(b)
                        fire_gather(b, j + b + NB)

            for b in range(min(NB, PH_ROWS)):
                wait_scatter(b)

        plsc.subcore_barrier()
        pltpu.sync_copy(agg_sh.at[pl.ds(s * seg, seg)], out_hbm.at[c, pl.ds(s * seg, seg)])

        @pl.when(s == NS - 1)
        def _():
            pltpu.sync_copy(agg_sh.at[pl.ds(NS * seg, N - NS * seg)],
                            out_hbm.at[c, pl.ds(NS * seg, N - NS * seg)])

    return agg_k(hn, ei5, zeros_rows)


def _prescale(h, deg_o):
    def body(h_ref, d_ref, o_ref):
        d = d_ref[...]
        cs = jnp.where(d > 0, lax.rsqrt(jnp.maximum(d, 1.0)), 1.0)
        o_ref[...] = h_ref[...] * cs

    return pl.pallas_call(
        body, out_shape=jax.ShapeDtypeStruct((N, DI), jnp.float32)
    )(h, deg_o)


def _dense_fused(parts, deg_i, h, W_f, b_f, W_p, b_p, g_f, bt_f, g_p, bt_p):
    P1 = N // BLK
    P2 = N // BLK
    M = BZ * KP
    RB = 800
    P3 = M // RB
    GPB = BLK // NPG
    inv_scale = 1.0 / math.sqrt(float(M))

    def body(p0_r, p1_r, d_r, h_r, wf_r, bf_r, wp_r, bp_r, gf_r, btf_r, gp_r,
             btp_r, o_r, of_s, op_s, pool_s, sf_s, sp_s):
        i = pl.program_id(0)

        @pl.when(i < P1)
        def _():
            d = d_r[...]
            cd = jnp.where(d > 0, lax.rsqrt(jnp.maximum(d, 1.0)), 1.0)
            agg = (p0_r[0] + p1_r[0]) * cd
            of = jnp.dot(agg, wf_r[...], preferred_element_type=jnp.float32) + bf_r[...]
            op = jnp.dot(agg, wp_r[...], preferred_element_type=jnp.float32) + bp_r[...]
            of_s[pl.ds(i * BLK, BLK), :] = of
            op_s[pl.ds(i * BLK, BLK), :] = op

            @pl.when(i == 0)
            def _():
                sf_s[...] = jnp.zeros_like(sf_s)
                sp_s[...] = jnp.zeros_like(sp_s)

            sf_s[...] += jnp.concatenate(
                [jnp.sum(of, axis=0, keepdims=True),
                 jnp.sum(of * of, axis=0, keepdims=True)], axis=0)
            sp_s[...] += jnp.concatenate(
                [jnp.sum(op, axis=0, keepdims=True),
                 jnp.sum(op * op, axis=0, keepdims=True)], axis=0)

        @pl.when(jnp.logical_and(i >= P1, i < P1 + P2))
        def _():
            ib = i - P1
            inv_n = 1.0 / N
            sf = sf_s[...]
            mean_f = sf[0:1] * inv_n
            var_f = sf[1:2] * inv_n - mean_f * mean_f
            of = of_s[pl.ds(ib * BLK, BLK), :]
            feat = (of - mean_f) * lax.rsqrt(var_f + 1e-5) * gf_r[...] + btf_r[...]
            feat = jnp.maximum(feat, 0.0) + h_r[...]
            sp_ = sp_s[...]
            mean_p = sp_[0:1] * inv_n
            var_p = sp_[1:2] * inv_n - mean_p * mean_p
            op = op_s[pl.ds(ib * BLK, BLK), :]
            a = (op - mean_p) * lax.rsqrt(var_p + 1e-5) * gp_r[...] + btp_r[...]
            a = jnp.maximum(a, 0.0)
            mx = jnp.max(a, axis=1, keepdims=True)
            ex = jnp.exp(a - mx)
            sm = ex / jnp.sum(ex, axis=1, keepdims=True)
            for g in range(GPB):
                ag = sm[g * NPG:(g + 1) * NPG]
                fg = feat[g * NPG:(g + 1) * NPG]
                pg = lax.dot_general(ag, fg, (((0,), (0,)), ((), ())),
                                     preferred_element_type=jnp.float32)
                pool_s[pl.ds(ib * GPB * KP + g * KP, KP), :] = pg.astype(jnp.bfloat16)

        @pl.when(i >= P1 + P2)
        def _():
            ob = i - (P1 + P2)
            pr = pool_s[pl.ds(ob * RB, RB), :]
            o_r[...] = lax.dot_general(
                pr, pool_s[...], (((1,), (1,)), ((), ())),
                preferred_element_type=jnp.float32) * inv_scale

    c0 = lambda i: (0, 0)
    return pl.pallas_call(
        body,
        grid=(P1 + P2 + P3,),
        in_specs=[
            pl.BlockSpec((1, BLK, DI), lambda i: (0, jnp.minimum(i, P1 - 1), 0)),
            pl.BlockSpec((1, BLK, DI), lambda i: (1, jnp.minimum(i, P1 - 1), 0)),
            pl.BlockSpec((BLK, 1), lambda i: (jnp.minimum(i, P1 - 1), 0)),
            pl.BlockSpec((BLK, DI),
                         lambda i: (jnp.clip(i - P1, 0, P2 - 1), 0)),
            pl.BlockSpec((DI, DI), c0),
            pl.BlockSpec((1, DI), c0),
            pl.BlockSpec((DI, KP), c0),
            pl.BlockSpec((1, KP), c0),
            pl.BlockSpec((1, DI), c0),
            pl.BlockSpec((1, DI), c0),
            pl.BlockSpec((1, KP), c0),
            pl.BlockSpec((1, KP), c0),
        ],
        out_specs=pl.BlockSpec((RB, M),
                               lambda i: (jnp.clip(i - P1 - P2, 0, P3 - 1), 0)),
        out_shape=jax.ShapeDtypeStruct((M, M), jnp.float32),
        scratch_shapes=[
            pltpu.VMEM((N, DI), jnp.float32),
            pltpu.VMEM((N, KP), jnp.float32),
            pltpu.VMEM((M, DI), jnp.bfloat16),
            pltpu.VMEM((2, DI), jnp.float32),
            pltpu.VMEM((2, KP), jnp.float32),
        ],
    )(parts, parts, deg_i, h, W_f, b_f, W_p, b_p, g_f, bt_f, g_p, bt_p)


def kernel(h, edge_index, e, W_feat, b_feat, gamma_feat, beta_feat,
           W_pool, b_pool, gamma_pool, beta_pool):
    del e
    ei4 = edge_index.reshape(2, NS, ER // NS, EC)
    ei5 = edge_index.reshape(2, 5 * NC * NS, 25, ECA)
    zeros_hist = jnp.zeros((N2,), jnp.float32)
    ones_ec = jnp.ones((EC,), jnp.float32)
    deg = _sc_degrees(ei4, zeros_hist, ones_ec)
    deg_o = deg[0, :N].reshape(N, 1)
    deg_i = deg[1, :N].reshape(N, 1)
    hn = _prescale(h, deg_o)
    zeros_rows = jnp.zeros((624, DI), jnp.float32)
    parts = _sc_agg(hn, ei5, zeros_rows)
    return _dense_fused(parts, deg_i, h, W_feat, b_feat.reshape(1, DI),
                        W_pool, b_pool.reshape(1, KP),
                        gamma_feat.reshape(1, DI), beta_feat.reshape(1, DI),
                        gamma_pool.reshape(1, KP), beta_pool.reshape(1, KP))
